# Initial kernel scaffold; baseline (speedup 1.0000x reference)
import jax
import jax.numpy as jnp
from jax.experimental import pallas as pl


def kernel(p3, p4, p5, cls_w, cls_b, reg_w, reg_b, w_cls, b_cls, w_reg, b_reg, w_ctn, b_ctn):
    raise NotImplementedError("write your pallas kernel here")


# speedup vs baseline: 11.7734x; 11.7734x over previous
import functools

import jax
import jax.numpy as jnp
from jax import lax
from jax.experimental import pallas as pl
import jax.experimental.pallas.tpu as pltpu

_NUM_CLASSES = 80
_D = 256
_IMG = 1024.0
_CONF = 0.05
_NMS_T = 0.6
_K = 1000
_NCAND = 3 * _K
_NPAD = 3072
_MM_DT = jnp.bfloat16


def _level_body(x_ref, cw_ref, cb_ref, rw_ref, rb_ref, hc_ref, hcb_ref,
                hr_ref, hrb_ref, boxes_ref, smax_ref, sarg_ref,
                *, R, W, H, stride):
    i = pl.program_id(0)
    rin = R + 8
    x = x_ref[pl.ds(i * R, rin), :, :]

    def conv_layer(xp, wl, bl, g0):
        ro = xp.shape[0] - 2
        xb = xp.astype(_MM_DT)
        acc = None
        for t in range(9):
            dy, dx = divmod(t, 3)
            xs = xb[dy:dy + ro, dx:dx + W, :].reshape(ro * W, _D)
            d = jnp.dot(xs, wl[t], preferred_element_type=jnp.float32)
            acc = d if acc is None else acc + d
        y = jnp.maximum(acc + bl[None, :], 0.0).reshape(ro, W, _D)
        rg = lax.broadcasted_iota(jnp.int32, (ro, 1, 1), 0) + g0
        y = jnp.where((rg >= 0) & (rg < H), y, 0.0)
        z = jnp.zeros((ro, 1, _D), jnp.float32)
        return jnp.concatenate([z, y, z], axis=1)

    cf = x
    rf = x
    for l in range(4):
        g0 = i * R - 3 + l
        cf = conv_layer(cf, cw_ref[l], cb_ref[l], g0)
        rf = conv_layer(rf, rw_ref[l], rb_ref[l], g0)

    cfb = cf[:, 1:1 + W, :].astype(_MM_DT).reshape(R * W, _D)
    rfb = rf[:, 1:1 + W, :].astype(_MM_DT).reshape(R * W, _D)
    cls = jnp.dot(cfb, hc_ref[...], preferred_element_type=jnp.float32) + hcb_ref[...][None, :]
    rc = jnp.dot(rfb, hr_ref[...], preferred_element_type=jnp.float32) + hrb_ref[...][None, :]

    ctn = rc[:, 4:5]
    scores = jnp.sqrt(jax.nn.sigmoid(cls) * jax.nn.sigmoid(ctn))
    smax_ref[...] = jnp.max(scores, axis=-1).reshape(R, W)
    sarg_ref[...] = jnp.argmax(scores, axis=-1).astype(jnp.int32).reshape(R, W)

    cy = (lax.broadcasted_iota(jnp.int32, (R, W), 0).astype(jnp.float32)
          + (i * R).astype(jnp.float32) + 0.5)
    cx = lax.broadcasted_iota(jnp.int32, (R, W), 1).astype(jnp.float32) + 0.5
    e0 = jnp.exp(rc[:, 0]).reshape(R, W)
    e1 = jnp.exp(rc[:, 1]).reshape(R, W)
    e2 = jnp.exp(rc[:, 2]).reshape(R, W)
    e3 = jnp.exp(rc[:, 3]).reshape(R, W)
    s = jnp.float32(stride)
    bx1 = jnp.clip((cx - e0) * s / _IMG, 0.0, 1.0)
    by1 = jnp.clip((cy - e1) * s / _IMG, 0.0, 1.0)
    bx2 = jnp.clip((cx + e2) * s / _IMG, 0.0, 1.0)
    by2 = jnp.clip((cy + e3) * s / _IMG, 0.0, 1.0)
    boxes_ref[...] = jnp.stack([bx1, by1, bx2, by2], axis=-1)


def _run_level(p, stride, cw, cb, rw, rb, hc, hcb, hr, hrb, R):
    H, W = p.shape[2], p.shape[3]
    x = jnp.transpose(p[0], (1, 2, 0))
    xpad = jnp.pad(x, ((4, 4), (1, 1), (0, 0)))
    nb = H // R
    body = functools.partial(_level_body, R=R, W=W, H=H, stride=float(stride))
    full = lambda shape: pl.BlockSpec(shape, lambda i: (0,) * len(shape))
    boxes, smax, sarg = pl.pallas_call(
        body,
        grid=(nb,),
        in_specs=[
            full((H + 8, W + 2, _D)),
            full((4, 9, _D, _D)), full((4, _D)),
            full((4, 9, _D, _D)), full((4, _D)),
            full((_D, _NUM_CLASSES)), full((_NUM_CLASSES,)),
            full((_D, 8)), full((8,)),
        ],
        out_specs=[
            pl.BlockSpec((R, W, 4), lambda i: (i, 0, 0)),
            pl.BlockSpec((R, W), lambda i: (i, 0)),
            pl.BlockSpec((R, W), lambda i: (i, 0)),
        ],
        out_shape=[
            jax.ShapeDtypeStruct((H, W, 4), jnp.float32),
            jax.ShapeDtypeStruct((H, W), jnp.float32),
            jax.ShapeDtypeStruct((H, W), jnp.int32),
        ],
        compiler_params=pltpu.CompilerParams(
            dimension_semantics=(pltpu.GridDimensionSemantics.PARALLEL,),
            vmem_limit_bytes=100 * 1024 * 1024,
        ),
    )(xpad, cw, cb, rw, rb, hc, hcb, hr, hrb)
    return (boxes.reshape(H * W, 4), smax.reshape(H * W), sarg.reshape(H * W))


def _nms_body(data_ref, dataT_ref, keep_ref, sorted_ref, sortedT_ref,
              supp_ref, rankc_ref):
    B = _NPAD
    nblk = B // 256

    s_row = dataT_ref[4:5, :]

    def blk(kb):
        return pl.multiple_of(kb * 256, 256)

    lane_i = lax.broadcasted_iota(jnp.int32, (1, B), 1).astype(jnp.float32)
    row_iota = lax.broadcasted_iota(jnp.int32, (256, B), 0).astype(jnp.float32)

    def rank_row_step(jb, rr):
        sj = data_ref[pl.ds(blk(jb), 256), 4:5]
        jg = row_iota + (jb * 256).astype(jnp.float32)
        gt = (sj > s_row) | ((sj == s_row) & (jg < lane_i))
        return rr + jnp.sum(jnp.where(gt, 1.0, 0.0), axis=0, keepdims=True)

    rank_row = lax.fori_loop(0, nblk, rank_row_step,
                             jnp.zeros((1, B), jnp.float32))

    lane_j = lax.broadcasted_iota(jnp.int32, (256, B), 1).astype(jnp.float32)

    def rank_col_step(ib, _):
        si = data_ref[pl.ds(blk(ib), 256), 4:5]
        ig = row_iota + (ib * 256).astype(jnp.float32)
        gt = (s_row > si) | ((s_row == si) & (lane_j < ig))
        rankc_ref[pl.ds(blk(ib), 256), :] = jnp.sum(
            jnp.where(gt, 1.0, 0.0), axis=1, keepdims=True)
        return 0

    lax.fori_loop(0, nblk, rank_col_step, 0)

    hp = jax.lax.Precision.HIGHEST

    def sort_step(kb, _):
        kr = row_iota + (kb * 256).astype(jnp.float32)
        Pk = jnp.where(rank_row == kr, 1.0, 0.0)
        sorted_ref[pl.ds(blk(kb), 256), :] = jnp.dot(
            Pk, data_ref[...], precision=hp, preferred_element_type=jnp.float32)
        return 0

    lax.fori_loop(0, nblk, sort_step, 0)

    kc_iota = lax.broadcasted_iota(jnp.int32, (B, 256), 1).astype(jnp.float32)

    def sortT_step(kb, _):
        kc = kc_iota + (kb * 256).astype(jnp.float32)
        PkT = jnp.where(rankc_ref[...] == kc, 1.0, 0.0)
        sortedT_ref[:, pl.ds(blk(kb), 256)] = jnp.dot(
            dataT_ref[...], PkT, precision=hp, preferred_element_type=jnp.float32)
        return 0

    lax.fori_loop(0, nblk, sortT_step, 0)

    x1r = sortedT_ref[0:1, :]
    y1r = sortedT_ref[1:2, :]
    x2r = sortedT_ref[2:3, :]
    y2r = sortedT_ref[3:4, :]
    lr = sortedT_ref[5:6, :]
    area_r = (x2r - x1r) * (y2r - y1r)
    keep = jnp.where(sortedT_ref[4:5, :] >= _CONF, 1.0, 0.0)

    col_iota = lax.broadcasted_iota(jnp.int32, (1, B), 1).astype(jnp.float32)
    sub_iota = lax.broadcasted_iota(jnp.int32, (8, 1), 0).astype(jnp.float32)

    def supp_block(kb, keep_v):
        base = blk(kb)
        x1c = sorted_ref[pl.ds(base, 256), 0:1]
        y1c = sorted_ref[pl.ds(base, 256), 1:2]
        x2c = sorted_ref[pl.ds(base, 256), 2:3]
        y2c = sorted_ref[pl.ds(base, 256), 3:4]
        lc = sorted_ref[pl.ds(base, 256), 5:6]
        area_c = (x2c - x1c) * (y2c - y1c)
        xx1 = jnp.maximum(x1c, x1r)
        yy1 = jnp.maximum(y1c, y1r)
        xx2 = jnp.minimum(x2c, x2r)
        yy2 = jnp.minimum(y2c, y2r)
        inter = jnp.maximum(1e-28, xx2 - xx1) * jnp.maximum(1e-28, yy2 - yy1)
        iou = inter / (area_c + area_r - inter + 1e-14)
        ig = row_iota + (kb * 256).astype(jnp.float32)
        later = col_iota > ig
        supp = (iou > _NMS_T) & (lc == lr) & later
        supp_ref[...] = jnp.where(supp, 1.0, 0.0).reshape(32, 8, B)

        def step(ii, kv):
            g = (kb * 256 + ii).astype(jnp.float32)
            oh = jnp.where(col_iota == g, 1.0, 0.0)
            kg = jnp.sum(kv * oh, axis=-1, keepdims=True)
            tile = supp_ref[pl.ds(ii // 8, 1), :, :].reshape(8, B)
            ohs = jnp.where(sub_iota == (ii % 8).astype(jnp.float32), 1.0, 0.0)
            srow = jnp.sum(tile * ohs, axis=0, keepdims=True)
            return kv * (1.0 - kg * srow)

        return lax.fori_loop(0, 256, step, keep_v)

    keep = lax.fori_loop(0, nblk, supp_block, keep)

    def scatter_step(jb, _):
        rc = rankc_ref[pl.ds(blk(jb), 256), :]
        Kj = jnp.where(rc == col_iota, 1.0, 0.0)
        keep_ref[pl.ds(blk(jb), 256), :] = jnp.sum(
            Kj * keep, axis=1, keepdims=True)
        return 0

    lax.fori_loop(0, nblk, scatter_step, 0)


def _run_nms(bboxes, scores, labels):
    n = bboxes.shape[0]
    data = jnp.zeros((_NPAD, 128), jnp.float32)
    data = data.at[:n, 0:4].set(bboxes)
    data = data.at[:, 4].set(jnp.pad(scores, (0, _NPAD - n),
                                     constant_values=-1.0))
    data = data.at[:n, 5].set(labels.astype(jnp.float32))
    dataT = jnp.transpose(data[:, :8], (1, 0))
    keep = pl.pallas_call(
        _nms_body,
        in_specs=[pl.BlockSpec(memory_space=pltpu.VMEM),
                  pl.BlockSpec(memory_space=pltpu.VMEM)],
        out_specs=pl.BlockSpec(memory_space=pltpu.VMEM),
        out_shape=jax.ShapeDtypeStruct((_NPAD, 1), jnp.float32),
        scratch_shapes=[
            pltpu.VMEM((_NPAD, 128), jnp.float32),
            pltpu.VMEM((8, _NPAD), jnp.float32),
            pltpu.VMEM((32, 8, _NPAD), jnp.float32),
            pltpu.VMEM((_NPAD, 1), jnp.float32),
        ],
        compiler_params=pltpu.CompilerParams(
            vmem_limit_bytes=100 * 1024 * 1024,
        ),
    )(data, dataT)
    return keep[:n, 0] > 0.5


def kernel(p3, p4, p5, cls_w, cls_b, reg_w, reg_b,
           w_cls, b_cls, w_reg, b_reg, w_ctn, b_ctn):
    cw = jnp.transpose(cls_w, (0, 3, 4, 2, 1)).reshape(4, 9, _D, _D).astype(_MM_DT)
    rw = jnp.transpose(reg_w, (0, 3, 4, 2, 1)).reshape(4, 9, _D, _D).astype(_MM_DT)
    hc = jnp.transpose(w_cls[:, :, 0, 0], (1, 0)).astype(_MM_DT)
    hr4 = jnp.transpose(w_reg[:, :, 0, 0], (1, 0))
    hr1 = jnp.transpose(w_ctn[:, :, 0, 0], (1, 0))
    hr = jnp.concatenate([hr4, hr1, jnp.zeros((_D, 3), jnp.float32)],
                         axis=1).astype(_MM_DT)
    hrb = jnp.concatenate([b_reg, b_ctn, jnp.zeros((3,), jnp.float32)])

    outs = []
    for p, stride, R in ((p3, 8, 32), (p4, 16, 32), (p5, 32, 16)):
        boxes, smax, sarg = _run_level(p, stride, cw, cls_b, rw, reg_b,
                                       hc, b_cls, hr, hrb, R)
        top_s, idx = lax.top_k(smax, _K)
        outs.append((boxes[idx], top_s, sarg[idx]))

    bboxes = jnp.concatenate([o[0] for o in outs], 0)
    scores = jnp.concatenate([o[1] for o in outs], 0)
    labels = jnp.concatenate([o[2] for o in outs], 0)
    keep = _run_nms(bboxes, scores, labels)
    return bboxes, scores, labels, keep
